# baseline (speedup 1.0000x reference)
import functools

import jax
import jax.numpy as jnp
from jax import lax
from jax.experimental import pallas as pl
from jax.experimental.pallas import tpu as pltpu
from jax.experimental.pallas import tpu_sc as plsc

N_IN = 100000
C = 128
N_OUT = 50000

CHUNK = 784
RNG_ROWS = 2 * CHUNK
LAST_B = N_OUT - 31 * RNG_ROWS - CHUNK
IDX_BLK = 2000
N_BLK = N_IN // IDX_BLK
VEC_PER_BLK = IDX_BLK // 16
UNROLL = 25
GB = 80
CAP = 2192
HSPILL = 100000
TRASH = CHUNK
NEG_INF = float("-inf")


def _body(inputs_hbm, vtmap_hbm, out_hbm, hspill_hbm, acc, idx_buf,
          pair_buf, abuf, bbuf, gidx, rows, gidx2, rows2, sem, sem2):
    cid = lax.axis_index("c")
    sid = lax.axis_index("s")
    wid = sid * 2 + cid
    lo = wid * RNG_ROWS
    hbase = wid * HSPILL

    zeros16 = jnp.zeros((16,), jnp.int32)
    neg16 = jnp.full((16,), NEG_INF, jnp.float32)
    lane = lax.iota(jnp.int32, 16)
    pfx_idx = [jnp.maximum(lane - kk, 0) for kk in (1, 2, 4, 8)]
    pfx_ge = [lane >= kk for kk in (1, 2, 4, 8)]
    target = lane + 1

    def _prefix16(m):
        x = jnp.where(m, jnp.int32(1), jnp.int32(0))
        for si, ge in zip(pfx_idx, pfx_ge):
            sh = x.at[si].get(mode="promise_in_bounds")
            x = x + jnp.where(ge, sh, jnp.int32(0))
        return x

    def _select_matched(cs):
        pos = zeros16
        for step in (8, 4, 2, 1):
            cand = pos + step
            cv = cs.at[cand - 1].get(mode="promise_in_bounds")
            pos = jnp.where(cv < target, cand, pos)
        return pos

    def _init_bufs(i, _):
        sl = pl.ds(i * 16, 16)
        pair_buf[sl] = zeros16
        abuf[sl] = zeros16
        bbuf[sl] = zeros16
        return 0
    lax.fori_loop(0, CAP // 16, _init_bufs, 0)

    def _init_acc(i, _):
        for v in range(8):
            acc[i, pl.ds(v * 16, 16)] = neg16
        return 0

    def _fin_acc(i, _):
        for v in range(8):
            sl = pl.ds(v * 16, 16)
            x = acc[i, sl]
            acc[i, sl] = jnp.where(x == NEG_INF, jnp.float32(0), x)
        return 0

    def _fire(src, boff, gi, rb, se):
        for v in range(GB // 16):
            code = src[pl.ds(boff + v * 16, 16)]
            gi[pl.ds(v * 16, 16)] = lax.shift_right_logical(code, 11)
        return pltpu.async_copy(inputs_hbm.at[gi], rb, se)

    def _apply_group(g16, tv, rbuf):
        ts = [tv[i] for i in range(16)]
        for i in range(16):
            t = ts[i]
            for v in range(8):
                sl = pl.ds(v * 16, 16)
                acc[t, sl] = jnp.maximum(acc[t, sl], rbuf[g16 + i, sl])
        return 0

    def _apply_batch(src, boff, rbuf, sub, valid_n):
        def _grp(g, _):
            g16 = g * 16
            code = src[pl.ds(boff + g16, 16)]
            tv = lax.bitwise_and(code, 2047) - sub
            ok = (g16 + lane) < valid_n
            _apply_group(g16, jnp.where(ok, tv, TRASH), rbuf)
            return 0
        ng = lax.div(valid_n + 15, 16)
        lax.fori_loop(0, ng, _grp, 0)
        return 0

    def _split_full(boff, pa, pb):
        for sv in range(GB // 16):
            codes = pair_buf[pl.ds(boff + sv * 16, 16)]
            tgt = lax.bitwise_and(codes, 2047)
            m_a = tgt < CHUNK
            cs_a = _prefix16(m_a)
            sel_a = _select_matched(cs_a)
            abuf[pl.ds(pa, 16)] = codes.at[sel_a].get(
                mode="promise_in_bounds")
            cn_a = cs_a[15]
            pa = pa + cn_a
            cs_b = target - cs_a
            sel_b = _select_matched(cs_b)
            bbuf[pl.ds(pb, 16)] = codes.at[sel_b].get(
                mode="promise_in_bounds")
            pb = pb + (16 - cn_a)
        return pa, pb

    def _move_front(buf, tail):
        for v in range(GB // 16):
            sl = pl.ds(v * 16, 16)
            buf[sl] = buf[pl.ds(tail + v * 16, 16)]

    def _move_front2(buf, tail):
        for v in range(2 * GB // 16):
            sl = pl.ds(v * 16, 16)
            buf[sl] = buf[pl.ds(tail + v * 16, 16)]

    def _flush_a_pairs(pa):
        npairs = lax.div(pa, 2 * GB)

        def _b(b, _):
            ba = 2 * b * GB
            ca = _fire(abuf, ba, gidx, rows, sem)
            cb = _fire(abuf, ba + GB, gidx2, rows2, sem2)
            ca.wait()
            _apply_batch(abuf, ba, rows, 0, jnp.int32(GB))
            cb.wait()
            _apply_batch(abuf, ba + GB, rows2, 0, jnp.int32(GB))
            return 0
        lax.fori_loop(0, npairs, _b, 0)
        tail = npairs * 2 * GB

        @pl.when(npairs > 0)
        def _():
            _move_front2(abuf, tail)
        return pa - tail

    def _flush_a_rest(pa):
        nfull = lax.div(pa, GB)

        @pl.when(nfull > 0)
        def _():
            _fire(abuf, 0, gidx, rows, sem).wait()
            _apply_batch(abuf, 0, rows, 0, jnp.int32(GB))
            _move_front(abuf, GB)
        return pa - nfull * GB

    def _spill_b_pairs(pb, soff):
        npairs = lax.div(pb, 2 * GB)

        def _b(b, _):
            ho = pl.multiple_of(hbase + soff + 2 * b * GB, 8)
            pltpu.sync_copy(bbuf.at[pl.ds(2 * b * GB, 2 * GB)],
                            hspill_hbm.at[pl.ds(ho, 2 * GB)])
            return 0
        lax.fori_loop(0, npairs, _b, 0)
        tail = npairs * 2 * GB

        @pl.when(npairs > 0)
        def _():
            _move_front2(bbuf, tail)
        return pb - tail, soff + tail

    def _spill_b_rest(pb, soff):
        nfull = lax.div(pb, GB)

        @pl.when(nfull > 0)
        def _():
            ho = pl.multiple_of(hbase + soff, 8)
            pltpu.sync_copy(bbuf.at[pl.ds(0, GB)],
                            hspill_hbm.at[pl.ds(ho, GB)])
            _move_front(bbuf, GB)
        return pb - nfull * GB, soff + nfull * GB

    lax.fori_loop(0, CHUNK + 1, _init_acc, 0)

    def _blk_body(blk, carry):
        pm, pa, pb, soff = carry
        pltpu.sync_copy(vtmap_hbm.at[pl.ds(blk * IDX_BLK, IDX_BLK)],
                        idx_buf)

        def _vec_body(kk, pm):
            comp, cnts = [], []
            for j in range(UNROLL):
                k = kk * UNROLL + j
                v = idx_buf[pl.ds(k * 16, 16)]
                d = v - lo
                m = (d >= 0) & (d < RNG_ROWS)
                cs = _prefix16(m)
                dcl = jnp.minimum(jnp.maximum(d, 0), 2047)
                gpos = blk * IDX_BLK + k * 16 + lane
                code = gpos * 2048 + dcl
                sel = _select_matched(cs)
                comp.append(code.at[sel].get(mode="promise_in_bounds"))
                cnts.append(cs[15])
            for j in range(UNROLL):
                pair_buf[pl.ds(pm, 16)] = comp[j]
                pm = pm + cnts[j]
            return pm

        pm = lax.fori_loop(0, VEC_PER_BLK // UNROLL, _vec_body, pm)

        nfull_m = lax.div(pm, GB)

        def _sp(b, c):
            return _split_full(b * GB, c[0], c[1])
        pa, pb = lax.fori_loop(0, nfull_m, _sp, (pa, pb))
        tail_m = nfull_m * GB

        @pl.when(nfull_m > 0)
        def _():
            _move_front(pair_buf, tail_m)
        pm = pm - tail_m

        pa = _flush_a_pairs(pa)
        pb, soff = _spill_b_pairs(pb, soff)
        return pm, pa, pb, soff

    pm, pa, pb, soff = lax.fori_loop(
        0, N_BLK, _blk_body,
        (jnp.int32(0), jnp.int32(0), jnp.int32(0), jnp.int32(0)))

    def _tail_split(sv, carry):
        pa, pb = carry
        codes = pair_buf[pl.ds(sv * 16, 16)]
        valid = (sv * 16 + lane) < pm
        tgt = lax.bitwise_and(codes, 2047)
        m_a = valid & (tgt < CHUNK)
        m_b = valid & (tgt >= CHUNK)
        cs_a = _prefix16(m_a)
        sel_a = _select_matched(cs_a)
        abuf[pl.ds(pa, 16)] = codes.at[sel_a].get(mode="promise_in_bounds")
        cs_b = _prefix16(m_b)
        sel_b = _select_matched(cs_b)
        bbuf[pl.ds(pb, 16)] = codes.at[sel_b].get(mode="promise_in_bounds")
        return pa + cs_a[15], pb + cs_b[15]

    ntv = lax.div(pm + 15, 16)
    pa, pb = lax.fori_loop(0, ntv, _tail_split, (pa, pb))
    pa = _flush_a_pairs(pa)
    pa = _flush_a_rest(pa)
    pb, soff = _spill_b_pairs(pb, soff)
    pb, soff = _spill_b_rest(pb, soff)

    @pl.when(pa > 0)
    def _():
        _fire(abuf, 0, gidx, rows, sem).wait()
        _apply_batch(abuf, 0, rows, 0, pa)

    lax.fori_loop(0, CHUNK, _fin_acc, 0)
    pltpu.sync_copy(acc.at[pl.ds(0, CHUNK)], out_hbm.at[pl.ds(lo, CHUNK)])

    lax.fori_loop(0, CHUNK + 1, _init_acc, 0)

    nspill = lax.div(soff, GB)
    npairs = lax.div(nspill, 2)

    def _pair_b(b, _):
        off = pl.multiple_of(hbase + 2 * b * GB, 8)
        pltpu.sync_copy(hspill_hbm.at[pl.ds(off, 2 * GB)],
                        pair_buf.at[pl.ds(0, 2 * GB)])
        ca = _fire(pair_buf, 0, gidx, rows, sem)
        cb = _fire(pair_buf, GB, gidx2, rows2, sem2)
        ca.wait()
        _apply_batch(pair_buf, 0, rows, CHUNK, jnp.int32(GB))
        cb.wait()
        _apply_batch(pair_buf, GB, rows2, CHUNK, jnp.int32(GB))
        return 0
    lax.fori_loop(0, npairs, _pair_b, 0)

    @pl.when(nspill != npairs * 2)
    def _():
        boff = pl.multiple_of(hbase + npairs * 2 * GB, 8)
        pltpu.sync_copy(hspill_hbm.at[pl.ds(boff, GB)],
                        pair_buf.at[pl.ds(0, GB)])
        _fire(pair_buf, 0, gidx, rows, sem).wait()
        _apply_batch(pair_buf, 0, rows, CHUNK, jnp.int32(GB))

    @pl.when(pb > 0)
    def _():
        _fire(bbuf, 0, gidx, rows, sem).wait()
        _apply_batch(bbuf, 0, rows, CHUNK, pb)

    lax.fori_loop(0, CHUNK, _fin_acc, 0)
    bo = lo + CHUNK

    @pl.when(wid == 31)
    def _():
        pltpu.sync_copy(acc.at[pl.ds(0, LAST_B)],
                        out_hbm.at[pl.ds(bo, LAST_B)])

    @pl.when(wid != 31)
    def _():
        pltpu.sync_copy(acc.at[pl.ds(0, CHUNK)],
                        out_hbm.at[pl.ds(bo, CHUNK)])


_pool = functools.partial(
    pl.kernel,
    out_type=[jax.ShapeDtypeStruct((N_OUT, C), jnp.float32),
              jax.ShapeDtypeStruct((32 * HSPILL,), jnp.int32)],
    mesh=plsc.VectorSubcoreMesh(core_axis_name="c", subcore_axis_name="s"),
    scratch_types=[
        pltpu.VMEM((CHUNK + 1, C), jnp.float32),
        pltpu.VMEM((IDX_BLK,), jnp.int32),
        pltpu.VMEM((CAP,), jnp.int32),
        pltpu.VMEM((CAP,), jnp.int32),
        pltpu.VMEM((CAP,), jnp.int32),
        pltpu.VMEM((GB,), jnp.int32),
        pltpu.VMEM((GB, C), jnp.float32),
        pltpu.VMEM((GB,), jnp.int32),
        pltpu.VMEM((GB, C), jnp.float32),
        pltpu.SemaphoreType.DMA,
        pltpu.SemaphoreType.DMA,
    ],
)(_body)


def kernel(inputs, vt_replace, vt_map, vt_out):
    del vt_replace, vt_out
    out, _ = _pool(inputs, vt_map.astype(jnp.int32))
    return out


# speedup vs baseline: 1.0368x; 1.0368x over previous
import functools

import jax
import jax.numpy as jnp
from jax import lax
from jax.experimental import pallas as pl
from jax.experimental.pallas import tpu as pltpu
from jax.experimental.pallas import tpu_sc as plsc

N_IN = 100000
C = 128
N_OUT = 50000

CHUNK = 784
RNG_ROWS = 2 * CHUNK
LAST_B = N_OUT - 31 * RNG_ROWS - CHUNK
IDX_BLK = 2000
N_BLK = N_IN // IDX_BLK
VEC_PER_BLK = IDX_BLK // 16
UNROLL = 25
GB = 80
CAP = 2096
HSPILL = 100000
TRASH = CHUNK
NEG_INF = float("-inf")


def _body(inputs_hbm, vtmap_hbm, out_hbm, hspill_hbm, acc, idx_buf,
          pair_buf, abuf, bbuf, gidx, rows, gidx2, rows2, sem, sem2):
    cid = lax.axis_index("c")
    sid = lax.axis_index("s")
    wid = sid * 2 + cid
    lo = wid * RNG_ROWS
    hbase = wid * HSPILL

    zeros16 = jnp.zeros((16,), jnp.int32)
    neg16 = jnp.full((16,), NEG_INF, jnp.float32)
    lane = lax.iota(jnp.int32, 16)
    pfx_idx = [jnp.maximum(lane - kk, 0) for kk in (1, 2, 4, 8)]
    pfx_ge = [lane >= kk for kk in (1, 2, 4, 8)]
    target = lane + 1

    def _prefix16(m):
        x = jnp.where(m, jnp.int32(1), jnp.int32(0))
        for si, ge in zip(pfx_idx, pfx_ge):
            sh = x.at[si].get(mode="promise_in_bounds")
            x = x + jnp.where(ge, sh, jnp.int32(0))
        return x

    def _select_matched(cs):
        pos = zeros16
        for step in (8, 4, 2, 1):
            cand = pos + step
            cv = cs.at[cand - 1].get(mode="promise_in_bounds")
            pos = jnp.where(cv < target, cand, pos)
        return pos

    def _init_bufs(i, _):
        sl = pl.ds(i * 16, 16)
        pair_buf[sl] = zeros16
        abuf[sl] = zeros16
        bbuf[sl] = zeros16
        return 0
    lax.fori_loop(0, CAP // 16, _init_bufs, 0)

    def _init_acc(i, _):
        for v in range(8):
            acc[i, pl.ds(v * 16, 16)] = neg16
        return 0

    def _fin_acc(i, _):
        for v in range(8):
            sl = pl.ds(v * 16, 16)
            x = acc[i, sl]
            acc[i, sl] = jnp.where(x == NEG_INF, jnp.float32(0), x)
        return 0

    def _fire(src, boff, gi, rb, se):
        for v in range(GB // 16):
            code = src[pl.ds(boff + v * 16, 16)]
            gi[pl.ds(v * 16, 16)] = lax.shift_right_logical(code, 11)
        return pltpu.async_copy(inputs_hbm.at[gi], rb, se)

    def _apply_group(g16, tv, rbuf):
        ts = [tv[i] for i in range(16)]
        for i in range(16):
            t = ts[i]
            for v in range(8):
                sl = pl.ds(v * 16, 16)
                acc[t, sl] = jnp.maximum(acc[t, sl], rbuf[g16 + i, sl])
        return 0

    def _apply_batch(src, boff, rbuf, sub, valid_n):
        def _grp(g, _):
            g16 = g * 16
            code = src[pl.ds(boff + g16, 16)]
            tv = lax.bitwise_and(code, 2047) - sub
            ok = (g16 + lane) < valid_n
            _apply_group(g16, jnp.where(ok, tv, TRASH), rbuf)
            return 0
        ng = lax.div(valid_n + 15, 16)
        lax.fori_loop(0, ng, _grp, 0)
        return 0

    def _split_full(boff, pa, pb):
        for sv in range(GB // 16):
            codes = pair_buf[pl.ds(boff + sv * 16, 16)]
            tgt = lax.bitwise_and(codes, 2047)
            m_a = tgt < CHUNK
            cs_a = _prefix16(m_a)
            sel_a = _select_matched(cs_a)
            abuf[pl.ds(pa, 16)] = codes.at[sel_a].get(
                mode="promise_in_bounds")
            cn_a = cs_a[15]
            pa = pa + cn_a
            cs_b = target - cs_a
            sel_b = _select_matched(cs_b)
            bbuf[pl.ds(pb, 16)] = codes.at[sel_b].get(
                mode="promise_in_bounds")
            pb = pb + (16 - cn_a)
        return pa, pb

    def _move_front(buf, tail):
        for v in range(GB // 16):
            sl = pl.ds(v * 16, 16)
            buf[sl] = buf[pl.ds(tail + v * 16, 16)]

    def _flush_a(pa):
        nfull = lax.div(pa, GB)

        def _b(b, _):
            _fire(abuf, b * GB, gidx, rows, sem).wait()
            _apply_batch(abuf, b * GB, rows, 0, jnp.int32(GB))
            return 0
        lax.fori_loop(0, nfull, _b, 0)
        tail = nfull * GB

        @pl.when(nfull > 0)
        def _():
            _move_front(abuf, tail)
        return pa - tail

    def _spill_b(pb, soff):
        nfull = lax.div(pb, GB)

        def _b(b, _):
            ho = pl.multiple_of(hbase + soff + b * GB, 8)
            pltpu.sync_copy(bbuf.at[pl.ds(b * GB, GB)],
                            hspill_hbm.at[pl.ds(ho, GB)])
            return 0
        lax.fori_loop(0, nfull, _b, 0)
        tail = nfull * GB

        @pl.when(nfull > 0)
        def _():
            _move_front(bbuf, tail)
        return pb - tail, soff + tail

    lax.fori_loop(0, CHUNK + 1, _init_acc, 0)

    def _blk_body(blk, carry):
        pm, pa, pb, soff = carry
        pltpu.sync_copy(vtmap_hbm.at[pl.ds(blk * IDX_BLK, IDX_BLK)],
                        idx_buf)

        def _vec_body(kk, pm):
            comp, cnts = [], []
            for j in range(UNROLL):
                k = kk * UNROLL + j
                v = idx_buf[pl.ds(k * 16, 16)]
                d = v - lo
                m = (d >= 0) & (d < RNG_ROWS)
                cs = _prefix16(m)
                dcl = jnp.minimum(jnp.maximum(d, 0), 2047)
                gpos = blk * IDX_BLK + k * 16 + lane
                code = gpos * 2048 + dcl
                sel = _select_matched(cs)
                comp.append(code.at[sel].get(mode="promise_in_bounds"))
                cnts.append(cs[15])
            for j in range(UNROLL):
                pair_buf[pl.ds(pm, 16)] = comp[j]
                pm = pm + cnts[j]
            return pm

        pm = lax.fori_loop(0, VEC_PER_BLK // UNROLL, _vec_body, pm)

        nfull_m = lax.div(pm, GB)

        def _sp(b, c):
            return _split_full(b * GB, c[0], c[1])
        pa, pb = lax.fori_loop(0, nfull_m, _sp, (pa, pb))
        tail_m = nfull_m * GB

        @pl.when(nfull_m > 0)
        def _():
            _move_front(pair_buf, tail_m)
        pm = pm - tail_m

        pa = _flush_a(pa)
        pb, soff = _spill_b(pb, soff)
        return pm, pa, pb, soff

    pm, pa, pb, soff = lax.fori_loop(
        0, N_BLK, _blk_body,
        (jnp.int32(0), jnp.int32(0), jnp.int32(0), jnp.int32(0)))

    def _tail_split(sv, carry):
        pa, pb = carry
        codes = pair_buf[pl.ds(sv * 16, 16)]
        valid = (sv * 16 + lane) < pm
        tgt = lax.bitwise_and(codes, 2047)
        m_a = valid & (tgt < CHUNK)
        m_b = valid & (tgt >= CHUNK)
        cs_a = _prefix16(m_a)
        sel_a = _select_matched(cs_a)
        abuf[pl.ds(pa, 16)] = codes.at[sel_a].get(mode="promise_in_bounds")
        cs_b = _prefix16(m_b)
        sel_b = _select_matched(cs_b)
        bbuf[pl.ds(pb, 16)] = codes.at[sel_b].get(mode="promise_in_bounds")
        return pa + cs_a[15], pb + cs_b[15]

    ntv = lax.div(pm + 15, 16)
    pa, pb = lax.fori_loop(0, ntv, _tail_split, (pa, pb))
    pa = _flush_a(pa)
    pb, soff = _spill_b(pb, soff)

    @pl.when(pa > 0)
    def _():
        _fire(abuf, 0, gidx, rows, sem).wait()
        _apply_batch(abuf, 0, rows, 0, pa)

    lax.fori_loop(0, CHUNK, _fin_acc, 0)
    pltpu.sync_copy(acc.at[pl.ds(0, CHUNK)], out_hbm.at[pl.ds(lo, CHUNK)])

    lax.fori_loop(0, CHUNK + 1, _init_acc, 0)

    nspill = lax.div(soff, GB)
    npairs = lax.div(nspill, 2)

    def _pair_b(b, _):
        off = pl.multiple_of(hbase + 2 * b * GB, 8)
        pltpu.sync_copy(hspill_hbm.at[pl.ds(off, 2 * GB)],
                        pair_buf.at[pl.ds(0, 2 * GB)])
        ca = _fire(pair_buf, 0, gidx, rows, sem)
        cb = _fire(pair_buf, GB, gidx2, rows2, sem2)
        ca.wait()
        _apply_batch(pair_buf, 0, rows, CHUNK, jnp.int32(GB))
        cb.wait()
        _apply_batch(pair_buf, GB, rows2, CHUNK, jnp.int32(GB))
        return 0
    lax.fori_loop(0, npairs, _pair_b, 0)

    @pl.when(nspill != npairs * 2)
    def _():
        boff = pl.multiple_of(hbase + npairs * 2 * GB, 8)
        pltpu.sync_copy(hspill_hbm.at[pl.ds(boff, GB)],
                        pair_buf.at[pl.ds(0, GB)])
        _fire(pair_buf, 0, gidx, rows, sem).wait()
        _apply_batch(pair_buf, 0, rows, CHUNK, jnp.int32(GB))

    @pl.when(pb > 0)
    def _():
        _fire(bbuf, 0, gidx, rows, sem).wait()
        _apply_batch(bbuf, 0, rows, CHUNK, pb)

    lax.fori_loop(0, CHUNK, _fin_acc, 0)
    bo = lo + CHUNK

    @pl.when(wid == 31)
    def _():
        pltpu.sync_copy(acc.at[pl.ds(0, LAST_B)],
                        out_hbm.at[pl.ds(bo, LAST_B)])

    @pl.when(wid != 31)
    def _():
        pltpu.sync_copy(acc.at[pl.ds(0, CHUNK)],
                        out_hbm.at[pl.ds(bo, CHUNK)])


_pool = functools.partial(
    pl.kernel,
    out_type=[jax.ShapeDtypeStruct((N_OUT, C), jnp.float32),
              jax.ShapeDtypeStruct((32 * HSPILL,), jnp.int32)],
    mesh=plsc.VectorSubcoreMesh(core_axis_name="c", subcore_axis_name="s"),
    scratch_types=[
        pltpu.VMEM((CHUNK + 1, C), jnp.float32),
        pltpu.VMEM((IDX_BLK,), jnp.int32),
        pltpu.VMEM((CAP,), jnp.int32),
        pltpu.VMEM((CAP,), jnp.int32),
        pltpu.VMEM((CAP,), jnp.int32),
        pltpu.VMEM((GB,), jnp.int32),
        pltpu.VMEM((GB, C), jnp.float32),
        pltpu.VMEM((GB,), jnp.int32),
        pltpu.VMEM((GB, C), jnp.float32),
        pltpu.SemaphoreType.DMA,
        pltpu.SemaphoreType.DMA,
    ],
)(_body)


def kernel(inputs, vt_replace, vt_map, vt_out):
    del vt_replace, vt_out
    out, _ = _pool(inputs, vt_map.astype(jnp.int32))
    return out
